# baseline (speedup 1.0000x reference)
import jax
import jax.numpy as jnp
import numpy as np
from jax import lax
from jax.experimental import pallas as pl
from jax.experimental.pallas import tpu as pltpu
from jax.experimental.pallas import tpu_sc as plsc

TOPK = 64
ROW_LEN = 32768
ROWS = 1024
NVREG = ROW_LEN // 16
ROWS_PER_W = ROWS // 32
MASK7F = np.int32(0x7FFFFFFF)
KEY_MIN = np.int32(-0x80000000)
UA = 8
UB = 4


def _key(x):
    b = plsc.bitcast(x, jnp.int32)
    return jnp.where(b < 0, b ^ MASK7F, b)


def _unkey(k):
    b = jnp.where(k < 0, k ^ MASK7F, k)
    return plsc.bitcast(b, jnp.float32)


def _bc(s, n=16):
    return lax.broadcast(s, (n,))


def _rev(x):
    return lax.rev(x, (0,))


def _sc_topk_body(x_hbm, vals_hbm, idx_hbm, row_v, cand_v, candk_v, hist_v,
                  wink_v, wini_v, outi_v, outv_v):
    wid = lax.axis_index("s") * 2 + lax.axis_index("c")
    lane = lax.iota(jnp.int32, 16)
    ones = jnp.ones((16,), jnp.int32)
    zeros16 = jnp.zeros((16,), jnp.int32)

    def count_cmp(thresh, n_grp, strict):
        ts = _bc(thresh)

        def cb(g, acc):
            for u in range(UB):
                k = candk_v[pl.ds(g * (16 * UB) + u * 16, 16)]
                m = (k > ts) if strict else (k >= ts)
                acc = acc + jnp.where(m, ones, zeros16)
            return acc

        acc = lax.fori_loop(0, n_grp, cb, zeros16)
        return jnp.sum(acc)

    def do_row(r, _):
        row = wid * ROWS_PER_W + r
        pltpu.sync_copy(x_hbm.at[pl.ds(row * ROW_LEN, ROW_LEN)], row_v)

        def zb(g, _):
            for u in range(UA):
                hist_v[pl.ds(g * (16 * UA) + u * 16, 16)] = zeros16
            return 0
        lax.fori_loop(0, 4096 // (16 * UA), zb, 0)

        def pa(g, _):
            for u in range(UA):
                x = row_v[pl.ds(g * (16 * UA) + u * 16, 16)]
                k = _key(x)
                digit = lax.shift_right_logical(k, 24) ^ 128
                hidx = (digit << 4) | lane
                plsc.addupdate_scatter(hist_v, [hidx], ones)
            return 0
        lax.fori_loop(0, NVREG // UA, pa, 0)

        for q in range(4):
            outv_v[pl.ds(q * 16, 16)] = row_v[pl.ds(q * 16, 16)]
            outi_v[pl.ds(q * 16, 16)] = hist_v[pl.ds(q * 16, 16)]
        pltpu.sync_copy(outv_v, vals_hbm.at[pl.ds(row * TOPK, TOPK)])
        pltpu.sync_copy(outi_v, idx_hbm.at[pl.ds(row * TOPK, TOPK)])
        return 0

    lax.fori_loop(0, ROWS_PER_W, do_row, 0)


@jax.jit
def _sc_topk(x_flat):
    f = pl.kernel(
        _sc_topk_body,
        out_type=[
            jax.ShapeDtypeStruct((ROWS * TOPK,), jnp.float32),
            jax.ShapeDtypeStruct((ROWS * TOPK,), jnp.int32),
        ],
        mesh=plsc.VectorSubcoreMesh(core_axis_name="c", subcore_axis_name="s",
                                    num_cores=2, num_subcores=16),
        scratch_types=[
            pltpu.VMEM((ROW_LEN,), jnp.float32),
            pltpu.VMEM((ROW_LEN,), jnp.int32),
            pltpu.VMEM((ROW_LEN + 16 * UB,), jnp.int32),
            pltpu.VMEM((4096,), jnp.int32),
            pltpu.VMEM((TOPK,), jnp.int32),
            pltpu.VMEM((TOPK,), jnp.int32),
            pltpu.VMEM((TOPK,), jnp.int32),
            pltpu.VMEM((TOPK,), jnp.float32),
        ],
        compiler_params=pltpu.CompilerParams(needs_layout_passes=False),
    )
    return f(x_flat)


def kernel(pool_score):
    b0, b1, n = pool_score.shape
    x_flat = pool_score.reshape(b0 * b1 * n)
    vals, idx = _sc_topk(x_flat)
    return (vals.reshape(b0, b1, TOPK), idx.reshape(b0, b1, TOPK))


# speedup vs baseline: 1.3708x; 1.3708x over previous
import jax
import jax.numpy as jnp
import numpy as np
from jax import lax
from jax.experimental import pallas as pl
from jax.experimental.pallas import tpu as pltpu
from jax.experimental.pallas import tpu_sc as plsc

TOPK = 64
ROW_LEN = 32768
ROWS = 1024
NVREG = ROW_LEN // 16
ROWS_PER_W = ROWS // 32
MASK7F = np.int32(0x7FFFFFFF)
KEY_MIN = np.int32(-0x80000000)
UA = 8
UB = 4


def _key(x):
    b = plsc.bitcast(x, jnp.int32)
    return jnp.where(b < 0, b ^ MASK7F, b)


def _unkey(k):
    b = jnp.where(k < 0, k ^ MASK7F, k)
    return plsc.bitcast(b, jnp.float32)


def _bc(s, n=16):
    return lax.broadcast(s, (n,))


def _rev(x):
    return lax.rev(x, (0,))


def _sc_topk_body(x_hbm, vals_hbm, idx_hbm, row_v, cand_v, candk_v, hist_v,
                  wink_v, wini_v, outi_v, outv_v):
    wid = lax.axis_index("s") * 2 + lax.axis_index("c")
    lane = lax.iota(jnp.int32, 16)
    ones = jnp.ones((16,), jnp.int32)
    zeros16 = jnp.zeros((16,), jnp.int32)

    def count_cmp(thresh, n_grp, strict):
        ts = _bc(thresh)

        @plsc.parallel_loop(0, n_grp * UB, unroll=UB, carry=zeros16)
        def cb(g, acc):
            k = candk_v[pl.ds(g * 16, 16)]
            m = (k > ts) if strict else (k >= ts)
            return acc + jnp.where(m, ones, zeros16)

        return jnp.sum(cb)

    def do_row(r, _):
        row = wid * ROWS_PER_W + r
        pltpu.sync_copy(x_hbm.at[pl.ds(row * ROW_LEN, ROW_LEN)], row_v)

        @plsc.parallel_loop(0, 256, unroll=UA)
        def zb(g):
            hist_v[pl.ds(g * 16, 16)] = zeros16

        @plsc.parallel_loop(0, NVREG, unroll=UA)
        def pa(j):
            x = row_v[pl.ds(j * 16, 16)]
            k = _key(x)
            digit = lax.shift_right_logical(k, 24) ^ 128
            hidx = (digit << 4) | lane
            plsc.addupdate_scatter(hist_v, [hidx], ones)

        def bscan(g, carry):
            b1, run, found = carry
            base = 255 - g * 16 - 15
            addr0 = (_bc(base) + lane) << 4
            tot = plsc.load_gather(hist_v, [addr0])
            for c in range(1, 16):
                tot = tot + plsc.load_gather(hist_v, [addr0 + _bc(jnp.int32(c))])
            suff = _rev(plsc.cumsum(_rev(tot)))
            crosses = (suff + _bc(run)) >= TOPK
            cnt = jnp.sum(jnp.where(crosses, ones, zeros16))
            hit = (~found) & (cnt > 0)
            b1 = jnp.where(hit, base + cnt - 1, b1)
            found = found | hit
            run = run + jnp.sum(tot)
            return b1, run, found

        b1, _, _ = lax.fori_loop(
            0, 16, bscan, (jnp.int32(0), jnp.int32(0), False))

        t0 = (b1 ^ 128) << 24
        t0s = _bc(t0)

        @plsc.parallel_loop(0, NVREG, unroll=UA, carry=zeros16)
        def pb(j, off_s):
            x = row_v[pl.ds(j * 16, 16)]
            k = _key(x)
            m = k >= t0s
            mi = jnp.where(m, ones, zeros16)
            pos = plsc.cumsum(mi) - 1 + off_s
            plsc.store_scatter(cand_v, [pos], j * 16 + lane, mask=m)
            plsc.store_scatter(candk_v, [pos], k, mask=m)
            return off_s + plsc.all_reduce_population_count(m)

        cand_n = jnp.max(pb)
        for u in range(UB):
            plsc.store_scatter(
                candk_v, [_bc(cand_n) + lane + 16 * u], _bc(KEY_MIN))

        n_grp = (cand_n + (16 * UB - 1)) // (16 * UB)

        def bs(i, t):
            bit = jnp.int32(1) << (jnp.int32(23) - i)
            tt = t | bit
            c = count_cmp(tt, n_grp, False)
            return jnp.where(c >= TOPK, tt, t)

        kstar = lax.fori_loop(0, 24, bs, t0)
        c_above = count_cmp(kstar, n_grp, True)
        k_eq = TOPK - c_above

        ks_s = _bc(kstar)
        cns = _bc(cand_n)
        nvr = (cand_n + 15) // 16

        k_eq_s = _bc(k_eq)

        @plsc.parallel_loop(0, nvr, carry=(zeros16, zeros16))
        def wc(v, carry):
            woff_s, eq_s = carry
            idxs = cand_v[pl.ds(v * 16, 16)]
            k = candk_v[pl.ds(v * 16, 16)]
            valid = (v * 16 + lane) < cns
            mA = (k > ks_s) & valid
            mB = (k == ks_s) & valid
            mBi = jnp.where(mB, ones, zeros16)
            csB = plsc.cumsum(mBi)
            mBsel = mB & ((csB + eq_s) <= k_eq_s)
            win = mA | mBsel
            wi = jnp.where(win, ones, zeros16)
            pos = plsc.cumsum(wi) - 1 + woff_s
            plsc.store_scatter(wini_v, [pos], idxs, mask=win)
            plsc.store_scatter(wink_v, [pos], k, mask=win)
            return (woff_s + plsc.all_reduce_population_count(win),
                    eq_s + plsc.all_reduce_population_count(mB))

        kv = tuple(wink_v[pl.ds(q * 16, 16)] for q in range(4))
        iv = tuple(wini_v[pl.ds(q * 16, 16)] for q in range(4))
        ok = (zeros16,) * 4
        oi = (zeros16,) * 4

        def ex(i, carry):
            kv, iv, ok, oi = carry
            mv = jnp.maximum(jnp.maximum(kv[0], kv[1]),
                             jnp.maximum(kv[2], kv[3]))
            mk = jnp.max(mv)
            mks = _bc(mk)
            big = _bc(jnp.int32(ROW_LEN))
            cands = [jnp.where(kv[q] == mks, iv[q], big) for q in range(4)]
            mi = jnp.min(jnp.minimum(jnp.minimum(cands[0], cands[1]),
                                     jnp.minimum(cands[2], cands[3])))
            mis = _bc(mi)
            i_s = _bc(i)
            slot = tuple((lane + 16 * q) == i_s for q in range(4))
            ok = tuple(jnp.where(slot[q], mks, ok[q]) for q in range(4))
            oi = tuple(jnp.where(slot[q], mis, oi[q]) for q in range(4))
            kv = tuple(
                jnp.where((kv[q] == mks) & (iv[q] == mis), _bc(KEY_MIN), kv[q])
                for q in range(4))
            return kv, iv, ok, oi

        _, _, ok, oi = lax.fori_loop(0, TOPK, ex, (kv, iv, ok, oi))

        for q in range(4):
            outv_v[pl.ds(q * 16, 16)] = _unkey(ok[q])
            outi_v[pl.ds(q * 16, 16)] = oi[q]
        pltpu.sync_copy(outv_v, vals_hbm.at[pl.ds(row * TOPK, TOPK)])
        pltpu.sync_copy(outi_v, idx_hbm.at[pl.ds(row * TOPK, TOPK)])
        return 0

    lax.fori_loop(0, ROWS_PER_W, do_row, 0)


@jax.jit
def _sc_topk(x_flat):
    f = pl.kernel(
        _sc_topk_body,
        out_type=[
            jax.ShapeDtypeStruct((ROWS * TOPK,), jnp.float32),
            jax.ShapeDtypeStruct((ROWS * TOPK,), jnp.int32),
        ],
        mesh=plsc.VectorSubcoreMesh(core_axis_name="c", subcore_axis_name="s",
                                    num_cores=2, num_subcores=16),
        scratch_types=[
            pltpu.VMEM((ROW_LEN,), jnp.float32),
            pltpu.VMEM((ROW_LEN,), jnp.int32),
            pltpu.VMEM((ROW_LEN + 16 * UB,), jnp.int32),
            pltpu.VMEM((4096,), jnp.int32),
            pltpu.VMEM((TOPK,), jnp.int32),
            pltpu.VMEM((TOPK,), jnp.int32),
            pltpu.VMEM((TOPK,), jnp.int32),
            pltpu.VMEM((TOPK,), jnp.float32),
        ],
        compiler_params=pltpu.CompilerParams(needs_layout_passes=False),
    )
    return f(x_flat)


def kernel(pool_score):
    b0, b1, n = pool_score.shape
    x_flat = pool_score.reshape(b0 * b1 * n)
    vals, idx = _sc_topk(x_flat)
    return (vals.reshape(b0, b1, TOPK), idx.reshape(b0, b1, TOPK))


# speedup vs baseline: 1.7600x; 1.2839x over previous
import jax
import jax.numpy as jnp
import numpy as np
from jax import lax
from jax.experimental import pallas as pl
from jax.experimental.pallas import tpu as pltpu
from jax.experimental.pallas import tpu_sc as plsc

TOPK = 64
ROW_LEN = 32768
ROWS = 1024
NVREG = ROW_LEN // 16
ROWS_PER_W = ROWS // 32
MASK7F = np.int32(0x7FFFFFFF)
KEY_MIN = np.int32(-0x80000000)
MARGIN_ROW = np.int32(0x400000)
MARGIN_BOOT = np.int32(0x800000)
UA = 8
UB = 4


def _key(x):
    b = plsc.bitcast(x, jnp.int32)
    return jnp.where(b < 0, b ^ MASK7F, b)


def _unkey(k):
    b = jnp.where(k < 0, k ^ MASK7F, k)
    return plsc.bitcast(b, jnp.float32)


def _bc(s, n=16):
    return lax.broadcast(s, (n,))


def _sc_topk_body(x_hbm, vals_hbm, idx_hbm, row_v, cand_v, candk_v,
                  wink_v, wini_v, outi_v, outv_v, dma_sem):
    wid = lax.axis_index("s") * 2 + lax.axis_index("c")
    lane = lax.iota(jnp.int32, 16)
    ones = jnp.ones((16,), jnp.int32)
    zeros16 = jnp.zeros((16,), jnp.int32)
    row0 = wid * ROWS_PER_W

    def row_copy(row):
        return pltpu.make_async_copy(
            x_hbm.at[pl.ds(row * ROW_LEN, ROW_LEN)], row_v, dma_sem)

    def count_cmp(thresh, n_grp, strict):
        ts = _bc(thresh)

        @plsc.parallel_loop(0, n_grp * UB, unroll=UB, carry=zeros16)
        def cb(g, acc):
            k = candk_v[pl.ds(g * 16, 16)]
            m = (k > ts) if strict else (k >= ts)
            return acc + jnp.where(m, ones, zeros16)

        return jnp.sum(cb)

    def do_row(r, t_pred):
        row = row0 + r
        row_copy(row).wait()

        def cc(c):
            return c[0] < TOPK

        def cbody(c):
            _, t_try = c
            ts = _bc(t_try)

            @plsc.parallel_loop(0, NVREG, unroll=UA, carry=zeros16)
            def pb(j, off_s):
                x = row_v[pl.ds(j * 16, 16)]
                k = _key(x)
                m = k >= ts
                mi = jnp.where(m, ones, zeros16)
                pos = plsc.cumsum(mi) - 1 + off_s
                plsc.store_scatter(cand_v, [pos], j * 16 + lane, mask=m)
                plsc.store_scatter(candk_v, [pos], k, mask=m)
                return off_s + plsc.all_reduce_population_count(m)

            n = jnp.max(pb)
            return n, jnp.where(n < TOPK, jnp.int32(KEY_MIN), t_try)

        cand_n, _ = lax.while_loop(cc, cbody, (jnp.int32(0), t_pred))

        nxt = jnp.where(r < ROWS_PER_W - 1, row + 1, row0)
        row_copy(nxt).start()

        for u in range(UB):
            plsc.store_scatter(
                candk_v, [_bc(cand_n) + lane + 16 * u], _bc(KEY_MIN))

        n_grp = (cand_n + (16 * UB - 1)) // (16 * UB)

        def bsearch(i, t):
            bit = jnp.int32(1) << (jnp.int32(31) - i)
            tt = t + bit
            c = count_cmp(tt, n_grp, False)
            return jnp.where(c >= TOPK, tt, t)

        kstar = lax.fori_loop(0, 32, bsearch, jnp.int32(KEY_MIN))
        c_above = count_cmp(kstar, n_grp, True)
        k_eq = TOPK - c_above

        ks_s = _bc(kstar)
        cns = _bc(cand_n)
        nvr = (cand_n + 15) // 16
        k_eq_s = _bc(k_eq)

        @plsc.parallel_loop(0, nvr, carry=(zeros16, zeros16))
        def wc(v, carry):
            woff_s, eq_s = carry
            idxs = cand_v[pl.ds(v * 16, 16)]
            k = candk_v[pl.ds(v * 16, 16)]
            valid = (v * 16 + lane) < cns
            mA = (k > ks_s) & valid
            mB = (k == ks_s) & valid
            mBi = jnp.where(mB, ones, zeros16)
            csB = plsc.cumsum(mBi)
            mBsel = mB & ((csB + eq_s) <= k_eq_s)
            win = mA | mBsel
            wi = jnp.where(win, ones, zeros16)
            pos = plsc.cumsum(wi) - 1 + woff_s
            plsc.store_scatter(wini_v, [pos], idxs, mask=win)
            plsc.store_scatter(wink_v, [pos], k, mask=win)
            return (woff_s + plsc.all_reduce_population_count(win),
                    eq_s + plsc.all_reduce_population_count(mB))

        kv = tuple(wink_v[pl.ds(q * 16, 16)] for q in range(4))
        iv = tuple(wini_v[pl.ds(q * 16, 16)] for q in range(4))
        ok = (zeros16,) * 4
        oi = (zeros16,) * 4

        def ex(i, carry):
            kv, iv, ok, oi = carry
            mv = jnp.maximum(jnp.maximum(kv[0], kv[1]),
                             jnp.maximum(kv[2], kv[3]))
            mk = jnp.max(mv)
            mks = _bc(mk)
            big = _bc(jnp.int32(ROW_LEN))
            cands = [jnp.where(kv[q] == mks, iv[q], big) for q in range(4)]
            mi = jnp.min(jnp.minimum(jnp.minimum(cands[0], cands[1]),
                                     jnp.minimum(cands[2], cands[3])))
            mis = _bc(mi)
            i_s = _bc(i)
            slot = tuple((lane + 16 * q) == i_s for q in range(4))
            ok = tuple(jnp.where(slot[q], mks, ok[q]) for q in range(4))
            oi = tuple(jnp.where(slot[q], mis, oi[q]) for q in range(4))
            kv = tuple(
                jnp.where((kv[q] == mks) & (iv[q] == mis), _bc(KEY_MIN), kv[q])
                for q in range(4))
            return kv, iv, ok, oi

        _, _, ok, oi = lax.fori_loop(0, TOPK, ex, (kv, iv, ok, oi))

        for q in range(4):
            outv_v[pl.ds(q * 16, 16)] = _unkey(ok[q])
            outi_v[pl.ds(q * 16, 16)] = oi[q]
        pltpu.sync_copy(outv_v, vals_hbm.at[pl.ds(row * TOPK, TOPK)])
        pltpu.sync_copy(outi_v, idx_hbm.at[pl.ds(row * TOPK, TOPK)])

        return jnp.where(kstar < KEY_MIN + MARGIN_ROW,
                         jnp.int32(KEY_MIN), kstar - MARGIN_ROW)

    row_copy(row0).start()
    row_copy(row0).wait()

    @plsc.parallel_loop(0, NVREG, unroll=UA, carry=_bc(jnp.int32(KEY_MIN)))
    def mx(j, m):
        return jnp.maximum(m, _key(row_v[pl.ds(j * 16, 16)]))

    maxk = jnp.max(mx)
    t0 = jnp.where(maxk < KEY_MIN + MARGIN_BOOT,
                   jnp.int32(KEY_MIN), maxk - MARGIN_BOOT)
    row_copy(row0).start()

    lax.fori_loop(0, ROWS_PER_W, do_row, t0)
    row_copy(row0).wait()


@jax.jit
def _sc_topk(x_flat):
    f = pl.kernel(
        _sc_topk_body,
        out_type=[
            jax.ShapeDtypeStruct((ROWS * TOPK,), jnp.float32),
            jax.ShapeDtypeStruct((ROWS * TOPK,), jnp.int32),
        ],
        mesh=plsc.VectorSubcoreMesh(core_axis_name="c", subcore_axis_name="s",
                                    num_cores=2, num_subcores=16),
        scratch_types=[
            pltpu.VMEM((ROW_LEN,), jnp.float32),
            pltpu.VMEM((ROW_LEN,), jnp.int32),
            pltpu.VMEM((ROW_LEN + 16 * UB,), jnp.int32),
            pltpu.VMEM((TOPK,), jnp.int32),
            pltpu.VMEM((TOPK,), jnp.int32),
            pltpu.VMEM((TOPK,), jnp.int32),
            pltpu.VMEM((TOPK,), jnp.float32),
            pltpu.SemaphoreType.DMA,
        ],
        compiler_params=pltpu.CompilerParams(needs_layout_passes=False),
    )
    return f(x_flat)


def kernel(pool_score):
    b0, b1, n = pool_score.shape
    x_flat = pool_score.reshape(b0 * b1 * n)
    vals, idx = _sc_topk(x_flat)
    return (vals.reshape(b0, b1, TOPK), idx.reshape(b0, b1, TOPK))


# speedup vs baseline: 2.2797x; 1.2953x over previous
import jax
import jax.numpy as jnp
import numpy as np
from jax import lax
from jax.experimental import pallas as pl
from jax.experimental.pallas import tpu as pltpu
from jax.experimental.pallas import tpu_sc as plsc

TOPK = 64
ROW_LEN = 32768
ROWS = 1024
NVREG = ROW_LEN // 16
ROWS_PER_W = ROWS // 32
MASK7F = np.int32(0x7FFFFFFF)
KEY_MIN = np.int32(-0x80000000)
MARGIN_ROW = np.int32(0x400000)
MARGIN_BOOT = np.int32(0x800000)
UA = 8
UB = 4


def _key(x):
    b = plsc.bitcast(x, jnp.int32)
    return jnp.where(b < 0, b ^ MASK7F, b)


def _unkey(k):
    b = jnp.where(k < 0, k ^ MASK7F, k)
    return plsc.bitcast(b, jnp.float32)


def _bc(s, n=16):
    return lax.broadcast(s, (n,))


def _sc_topk_body(x_hbm, vals_hbm, idx_hbm, row_v, cand_v, candk_v,
                  wink_v, wini_v, outi_v, outv_v, dma_sem):
    wid = lax.axis_index("s") * 2 + lax.axis_index("c")
    lane = lax.iota(jnp.int32, 16)
    ones = jnp.ones((16,), jnp.int32)
    zeros16 = jnp.zeros((16,), jnp.int32)
    row0 = wid * ROWS_PER_W

    def row_copy(row):
        return pltpu.make_async_copy(x_hbm.at[row], row_v, dma_sem)

    def count_cmp(thresh, n_grp, strict):
        ts = _bc(thresh)

        @plsc.parallel_loop(0, n_grp * UB, unroll=UB, carry=zeros16)
        def cb(g, acc):
            k = candk_v[pl.ds(g * 16, 16)]
            m = (k > ts) if strict else (k >= ts)
            return acc + jnp.where(m, ones, zeros16)

        return jnp.sum(cb)

    def do_row(r, t_pred):
        row = row0 + r
        row_copy(row).wait()

        def cc(c):
            return c[0] < TOPK

        def cbody(c):
            _, t_try = c
            ts = _bc(t_try)

            @plsc.parallel_loop(0, NVREG, unroll=UA, carry=zeros16)
            def pb(j, off_s):
                x = row_v[pl.ds(j * 16, 16)]
                k = _key(x)
                m = k >= ts
                mi = jnp.where(m, ones, zeros16)
                pos = plsc.cumsum(mi) - 1 + off_s
                plsc.store_scatter(cand_v, [pos], j * 16 + lane, mask=m)
                plsc.store_scatter(candk_v, [pos], k, mask=m)
                return off_s + plsc.all_reduce_population_count(m)

            n = jnp.max(pb)
            return n, jnp.where(n < TOPK, jnp.int32(KEY_MIN), t_try)

        cand_n, _ = lax.while_loop(cc, cbody, (jnp.int32(0), t_pred))

        nxt = jnp.where(r < ROWS_PER_W - 1, row + 1, row0)
        row_copy(nxt).start()

        for u in range(UB):
            plsc.store_scatter(
                candk_v, [_bc(cand_n) + lane + 16 * u], _bc(KEY_MIN))

        n_grp = (cand_n + (16 * UB - 1)) // (16 * UB)

        def bsearch(i, t):
            bit = jnp.int32(1) << (jnp.int32(31) - i)
            tt = t + bit
            c = count_cmp(tt, n_grp, False)
            return jnp.where(c >= TOPK, tt, t)

        kstar = lax.fori_loop(0, 32, bsearch, jnp.int32(KEY_MIN))
        c_above = count_cmp(kstar, n_grp, True)
        k_eq = TOPK - c_above

        ks_s = _bc(kstar)
        cns = _bc(cand_n)
        nvr = (cand_n + 15) // 16
        k_eq_s = _bc(k_eq)

        @plsc.parallel_loop(0, nvr, carry=(zeros16, zeros16))
        def wc(v, carry):
            woff_s, eq_s = carry
            idxs = cand_v[pl.ds(v * 16, 16)]
            k = candk_v[pl.ds(v * 16, 16)]
            valid = (v * 16 + lane) < cns
            mA = (k > ks_s) & valid
            mB = (k == ks_s) & valid
            mBi = jnp.where(mB, ones, zeros16)
            csB = plsc.cumsum(mBi)
            mBsel = mB & ((csB + eq_s) <= k_eq_s)
            win = mA | mBsel
            wi = jnp.where(win, ones, zeros16)
            pos = plsc.cumsum(wi) - 1 + woff_s
            plsc.store_scatter(wini_v, [pos], idxs, mask=win)
            plsc.store_scatter(wink_v, [pos], k, mask=win)
            return (woff_s + plsc.all_reduce_population_count(win),
                    eq_s + plsc.all_reduce_population_count(mB))

        kv = tuple(wink_v[pl.ds(q * 16, 16)] for q in range(4))
        iv = tuple(wini_v[pl.ds(q * 16, 16)] for q in range(4))
        ok = (zeros16,) * 4
        oi = (zeros16,) * 4

        def ex(i, carry):
            kv, iv, ok, oi = carry
            mv = jnp.maximum(jnp.maximum(kv[0], kv[1]),
                             jnp.maximum(kv[2], kv[3]))
            mk = jnp.max(mv)
            mks = _bc(mk)
            big = _bc(jnp.int32(ROW_LEN))
            cands = [jnp.where(kv[q] == mks, iv[q], big) for q in range(4)]
            mi = jnp.min(jnp.minimum(jnp.minimum(cands[0], cands[1]),
                                     jnp.minimum(cands[2], cands[3])))
            mis = _bc(mi)
            i_s = _bc(i)
            slot = tuple((lane + 16 * q) == i_s for q in range(4))
            ok = tuple(jnp.where(slot[q], mks, ok[q]) for q in range(4))
            oi = tuple(jnp.where(slot[q], mis, oi[q]) for q in range(4))
            kv = tuple(
                jnp.where((kv[q] == mks) & (iv[q] == mis), _bc(KEY_MIN), kv[q])
                for q in range(4))
            return kv, iv, ok, oi

        _, _, ok, oi = lax.fori_loop(0, TOPK, ex, (kv, iv, ok, oi))

        for q in range(4):
            outv_v[pl.ds(q * 16, 16)] = _unkey(ok[q])
            outi_v[pl.ds(q * 16, 16)] = oi[q]
        pltpu.sync_copy(outv_v, vals_hbm.at[row])
        pltpu.sync_copy(outi_v, idx_hbm.at[row])

        return jnp.where(kstar < KEY_MIN + MARGIN_ROW,
                         jnp.int32(KEY_MIN), kstar - MARGIN_ROW)

    row_copy(row0).start()
    row_copy(row0).wait()

    @plsc.parallel_loop(0, NVREG, unroll=UA, carry=_bc(jnp.int32(KEY_MIN)))
    def mx(j, m):
        return jnp.maximum(m, _key(row_v[pl.ds(j * 16, 16)]))

    maxk = jnp.max(mx)
    t0 = jnp.where(maxk < KEY_MIN + MARGIN_BOOT,
                   jnp.int32(KEY_MIN), maxk - MARGIN_BOOT)
    row_copy(row0).start()

    lax.fori_loop(0, ROWS_PER_W, do_row, t0)
    row_copy(row0).wait()


@jax.jit
def _sc_topk(x_flat):
    f = pl.kernel(
        _sc_topk_body,
        out_type=[
            jax.ShapeDtypeStruct((ROWS, TOPK), jnp.float32),
            jax.ShapeDtypeStruct((ROWS, TOPK), jnp.int32),
        ],
        mesh=plsc.VectorSubcoreMesh(core_axis_name="c", subcore_axis_name="s",
                                    num_cores=2, num_subcores=16),
        scratch_types=[
            pltpu.VMEM((ROW_LEN,), jnp.float32),
            pltpu.VMEM((ROW_LEN,), jnp.int32),
            pltpu.VMEM((ROW_LEN + 16 * UB,), jnp.int32),
            pltpu.VMEM((TOPK,), jnp.int32),
            pltpu.VMEM((TOPK,), jnp.int32),
            pltpu.VMEM((TOPK,), jnp.int32),
            pltpu.VMEM((TOPK,), jnp.float32),
            pltpu.SemaphoreType.DMA,
        ],
        compiler_params=pltpu.CompilerParams(needs_layout_passes=False),
    )
    return f(x_flat)


def kernel(pool_score):
    b0, b1, n = pool_score.shape
    x2 = pool_score.reshape(b0 * b1, n)
    vals, idx = _sc_topk(x2)
    return (vals.reshape(b0, b1, TOPK), idx.reshape(b0, b1, TOPK))


# speedup vs baseline: 2.4353x; 1.0682x over previous
import jax
import jax.numpy as jnp
import numpy as np
from jax import lax
from jax.experimental import pallas as pl
from jax.experimental.pallas import tpu as pltpu
from jax.experimental.pallas import tpu_sc as plsc

TOPK = 64
ROW_LEN = 32768
ROWS = 1024
NVREG = ROW_LEN // 16
ROWS_PER_W = ROWS // 32
MASK7F = np.int32(0x7FFFFFFF)
KEY_MIN = np.int32(-0x80000000)
MARGIN_ROW = np.int32(0x400000)
MARGIN_BOOT = np.int32(0x800000)
KEY_NEG_INF = np.int32(-2139095041)
UA = 8
UB = 4


def _key(x):
    b = plsc.bitcast(x, jnp.int32)
    return jnp.where(b < 0, b ^ MASK7F, b)


def _unkey(k):
    b = jnp.where(k < 0, k ^ MASK7F, k)
    return plsc.bitcast(b, jnp.float32)


def _bc(s, n=16):
    return lax.broadcast(s, (n,))


def _sc_topk_body(x_hbm, vals_hbm, idx_hbm, row_v, cand_v, candk_v,
                  wink_v, wini_v, outi_v, outv_v, dma_sem):
    wid = lax.axis_index("s") * 2 + lax.axis_index("c")
    lane = lax.iota(jnp.int32, 16)
    ones = jnp.ones((16,), jnp.int32)
    zeros16 = jnp.zeros((16,), jnp.int32)
    row0 = wid * ROWS_PER_W

    def row_copy(row):
        return pltpu.make_async_copy(x_hbm.at[row], row_v, dma_sem)

    def count_cmp(thresh, n_grp, strict):
        ts = _bc(thresh)

        @plsc.parallel_loop(0, n_grp * UB, unroll=UB, carry=zeros16)
        def cb(g, acc):
            k = candk_v[pl.ds(g * 16, 16)]
            m = (k > ts) if strict else (k >= ts)
            return acc + jnp.where(m, ones, zeros16)

        return jnp.sum(cb)

    def do_row(r, t_pred):
        row = row0 + r
        row_copy(row).wait()

        def cc(c):
            return c[0] < TOPK

        def cbody(c):
            _, t_try = c
            tf = _unkey(_bc(jnp.maximum(t_try, KEY_NEG_INF)))

            @plsc.parallel_loop(0, NVREG, unroll=UA, carry=zeros16)
            def pb(j, off_s):
                x = row_v[pl.ds(j * 16, 16)]
                m = x >= tf
                mi = jnp.where(m, ones, zeros16)
                pos = plsc.cumsum(mi) - 1 + off_s
                plsc.store_scatter(cand_v, [pos], j * 16 + lane, mask=m)
                plsc.store_scatter(candk_v, [pos], plsc.bitcast(x, jnp.int32),
                                   mask=m)
                return off_s + plsc.all_reduce_population_count(m)

            n = jnp.max(pb)
            return n, jnp.where(n < TOPK, jnp.int32(KEY_MIN), t_try)

        cand_n, _ = lax.while_loop(cc, cbody, (jnp.int32(0), t_pred))

        nxt = jnp.where(r < ROWS_PER_W - 1, row + 1, row0)
        row_copy(nxt).start()

        n_grp = (cand_n + (16 * UB - 1)) // (16 * UB)

        @plsc.parallel_loop(0, n_grp * UB, unroll=UB, carry=jnp.int32(0))
        def kf(v, acc):
            b = candk_v[pl.ds(v * 16, 16)]
            candk_v[pl.ds(v * 16, 16)] = jnp.where(b < 0, b ^ MASK7F, b)
            return acc

        for u in range(UB):
            plsc.store_scatter(
                candk_v, [_bc(cand_n) + lane + 16 * u], _bc(KEY_MIN))

        def bsearch(i, t):
            bit = jnp.int32(1) << (jnp.int32(31) - i)
            tt = t + bit
            c = count_cmp(tt, n_grp, False)
            return jnp.where(c >= TOPK, tt, t)

        kstar = lax.fori_loop(0, 32, bsearch, jnp.int32(KEY_MIN))
        c_above = count_cmp(kstar, n_grp, True)
        k_eq = TOPK - c_above

        ks_s = _bc(kstar)
        cns = _bc(cand_n)
        nvr = (cand_n + 15) // 16
        k_eq_s = _bc(k_eq)

        @plsc.parallel_loop(0, nvr, carry=(zeros16, zeros16))
        def wc(v, carry):
            woff_s, eq_s = carry
            idxs = cand_v[pl.ds(v * 16, 16)]
            k = candk_v[pl.ds(v * 16, 16)]
            valid = (v * 16 + lane) < cns
            mA = (k > ks_s) & valid
            mB = (k == ks_s) & valid
            mBi = jnp.where(mB, ones, zeros16)
            csB = plsc.cumsum(mBi)
            mBsel = mB & ((csB + eq_s) <= k_eq_s)
            win = mA | mBsel
            wi = jnp.where(win, ones, zeros16)
            pos = plsc.cumsum(wi) - 1 + woff_s
            plsc.store_scatter(wini_v, [pos], idxs, mask=win)
            plsc.store_scatter(wink_v, [pos], k, mask=win)
            return (woff_s + plsc.all_reduce_population_count(win),
                    eq_s + plsc.all_reduce_population_count(mB))

        kv = tuple(wink_v[pl.ds(q * 16, 16)] for q in range(4))
        iv = tuple(wini_v[pl.ds(q * 16, 16)] for q in range(4))
        ok = (zeros16,) * 4
        oi = (zeros16,) * 4

        def ex(i, carry):
            kv, iv, ok, oi = carry
            mv = jnp.maximum(jnp.maximum(kv[0], kv[1]),
                             jnp.maximum(kv[2], kv[3]))
            mk = jnp.max(mv)
            mks = _bc(mk)
            big = _bc(jnp.int32(ROW_LEN))
            cands = [jnp.where(kv[q] == mks, iv[q], big) for q in range(4)]
            mi = jnp.min(jnp.minimum(jnp.minimum(cands[0], cands[1]),
                                     jnp.minimum(cands[2], cands[3])))
            mis = _bc(mi)
            i_s = _bc(i)
            slot = tuple((lane + 16 * q) == i_s for q in range(4))
            ok = tuple(jnp.where(slot[q], mks, ok[q]) for q in range(4))
            oi = tuple(jnp.where(slot[q], mis, oi[q]) for q in range(4))
            kv = tuple(
                jnp.where((kv[q] == mks) & (iv[q] == mis), _bc(KEY_MIN), kv[q])
                for q in range(4))
            return kv, iv, ok, oi

        _, _, ok, oi = lax.fori_loop(0, TOPK, ex, (kv, iv, ok, oi))

        for q in range(4):
            outv_v[pl.ds(q * 16, 16)] = _unkey(ok[q])
            outi_v[pl.ds(q * 16, 16)] = oi[q]
        pltpu.sync_copy(outv_v, vals_hbm.at[row])
        pltpu.sync_copy(outi_v, idx_hbm.at[row])

        return jnp.where(kstar < KEY_MIN + MARGIN_ROW,
                         jnp.int32(KEY_MIN), kstar - MARGIN_ROW)

    row_copy(row0).start()
    row_copy(row0).wait()

    @plsc.parallel_loop(0, NVREG, unroll=UA, carry=_bc(jnp.int32(KEY_MIN)))
    def mx(j, m):
        return jnp.maximum(m, _key(row_v[pl.ds(j * 16, 16)]))

    maxk = jnp.max(mx)
    t0 = jnp.where(maxk < KEY_MIN + MARGIN_BOOT,
                   jnp.int32(KEY_MIN), maxk - MARGIN_BOOT)
    row_copy(row0).start()

    lax.fori_loop(0, ROWS_PER_W, do_row, t0)
    row_copy(row0).wait()


@jax.jit
def _sc_topk(x_flat):
    f = pl.kernel(
        _sc_topk_body,
        out_type=[
            jax.ShapeDtypeStruct((ROWS, TOPK), jnp.float32),
            jax.ShapeDtypeStruct((ROWS, TOPK), jnp.int32),
        ],
        mesh=plsc.VectorSubcoreMesh(core_axis_name="c", subcore_axis_name="s",
                                    num_cores=2, num_subcores=16),
        scratch_types=[
            pltpu.VMEM((ROW_LEN,), jnp.float32),
            pltpu.VMEM((ROW_LEN,), jnp.int32),
            pltpu.VMEM((ROW_LEN + 16 * UB,), jnp.int32),
            pltpu.VMEM((TOPK,), jnp.int32),
            pltpu.VMEM((TOPK,), jnp.int32),
            pltpu.VMEM((TOPK,), jnp.int32),
            pltpu.VMEM((TOPK,), jnp.float32),
            pltpu.SemaphoreType.DMA,
        ],
        compiler_params=pltpu.CompilerParams(needs_layout_passes=False),
    )
    return f(x_flat)


def kernel(pool_score):
    b0, b1, n = pool_score.shape
    x2 = pool_score.reshape(b0 * b1, n)
    vals, idx = _sc_topk(x2)
    return (vals.reshape(b0, b1, TOPK), idx.reshape(b0, b1, TOPK))


# speedup vs baseline: 2.5387x; 1.0424x over previous
import jax
import jax.numpy as jnp
import numpy as np
from jax import lax
from jax.experimental import pallas as pl
from jax.experimental.pallas import tpu as pltpu
from jax.experimental.pallas import tpu_sc as plsc

TOPK = 64
ROW_LEN = 32768
ROWS = 1024
NVREG = ROW_LEN // 16
ROWS_PER_W = ROWS // 32
MASK7F = np.int32(0x7FFFFFFF)
KEY_MIN = np.int32(-0x80000000)
MARGIN_ROW = np.int32(0x400000)
MARGIN_BOOT = np.int32(0x800000)
KEY_NEG_INF = np.int32(-2139095041)
UA = 8
UB = 4


def _key(x):
    b = plsc.bitcast(x, jnp.int32)
    return jnp.where(b < 0, b ^ MASK7F, b)


def _unkey(k):
    b = jnp.where(k < 0, k ^ MASK7F, k)
    return plsc.bitcast(b, jnp.float32)


def _bc(s, n=16):
    return lax.broadcast(s, (n,))


def _sc_topk_body(x_hbm, vals_hbm, idx_hbm, row_v, cand_v, candk_v,
                  wink_v, wini_v, outi_v, outv_v, dma_sem):
    wid = lax.axis_index("s") * 2 + lax.axis_index("c")
    lane = lax.iota(jnp.int32, 16)
    ones = jnp.ones((16,), jnp.int32)
    zeros16 = jnp.zeros((16,), jnp.int32)
    row0 = wid * ROWS_PER_W

    def row_copy(row):
        return pltpu.make_async_copy(x_hbm.at[row], row_v, dma_sem)

    def count_cmp(thresh, n_grp, strict):
        ts = _bc(thresh)

        @plsc.parallel_loop(0, n_grp * UB, unroll=UB, carry=zeros16)
        def cb(g, acc):
            k = candk_v[pl.ds(g * 16, 16)]
            m = (k > ts) if strict else (k >= ts)
            return acc + jnp.where(m, ones, zeros16)

        return jnp.sum(cb)

    def do_row(r, t_pred):
        row = row0 + r
        row_copy(row).wait()

        def cc(c):
            return c[0] < TOPK

        def cbody(c):
            _, t_try = c
            tf = _unkey(_bc(jnp.maximum(t_try, KEY_NEG_INF)))

            @plsc.parallel_loop(0, NVREG, unroll=UA, carry=zeros16)
            def pb(j, off_s):
                x = row_v[pl.ds(j * 16, 16)]
                m = x >= tf
                mi = jnp.where(m, ones, zeros16)
                pos = plsc.cumsum(mi) - 1 + off_s
                plsc.store_scatter(cand_v, [pos], j * 16 + lane, mask=m)
                return off_s + plsc.all_reduce_population_count(m)

            n = jnp.max(pb)
            return n, jnp.where(n < TOPK, jnp.int32(KEY_MIN), t_try)

        cand_n, _ = lax.while_loop(cc, cbody, (jnp.int32(0), t_pred))

        n_grp = (cand_n + (16 * UB - 1)) // (16 * UB)

        for u in range(UB):
            plsc.store_scatter(cand_v, [_bc(cand_n) + lane + 16 * u], zeros16)

        @plsc.parallel_loop(0, n_grp * UB, unroll=UB, carry=jnp.int32(0))
        def kf(v, acc):
            idx = cand_v[pl.ds(v * 16, 16)]
            b = plsc.bitcast(plsc.load_gather(row_v, [idx]), jnp.int32)
            candk_v[pl.ds(v * 16, 16)] = jnp.where(b < 0, b ^ MASK7F, b)
            return acc

        nxt = jnp.where(r < ROWS_PER_W - 1, row + 1, row0)
        row_copy(nxt).start()

        for u in range(UB):
            plsc.store_scatter(
                candk_v, [_bc(cand_n) + lane + 16 * u], _bc(KEY_MIN))

        def bsearch(i, t):
            bit = jnp.int32(1) << (jnp.int32(31) - i)
            tt = t + bit
            c = count_cmp(tt, n_grp, False)
            return jnp.where(c >= TOPK, tt, t)

        kstar = lax.fori_loop(0, 32, bsearch, jnp.int32(KEY_MIN))
        c_above = count_cmp(kstar, n_grp, True)
        k_eq = TOPK - c_above

        ks_s = _bc(kstar)
        cns = _bc(cand_n)
        nvr = (cand_n + 15) // 16
        k_eq_s = _bc(k_eq)

        @plsc.parallel_loop(0, nvr, carry=(zeros16, zeros16))
        def wc(v, carry):
            woff_s, eq_s = carry
            idxs = cand_v[pl.ds(v * 16, 16)]
            k = candk_v[pl.ds(v * 16, 16)]
            valid = (v * 16 + lane) < cns
            mA = (k > ks_s) & valid
            mB = (k == ks_s) & valid
            mBi = jnp.where(mB, ones, zeros16)
            csB = plsc.cumsum(mBi)
            mBsel = mB & ((csB + eq_s) <= k_eq_s)
            win = mA | mBsel
            wi = jnp.where(win, ones, zeros16)
            pos = plsc.cumsum(wi) - 1 + woff_s
            plsc.store_scatter(wini_v, [pos], idxs, mask=win)
            plsc.store_scatter(wink_v, [pos], k, mask=win)
            return (woff_s + plsc.all_reduce_population_count(win),
                    eq_s + plsc.all_reduce_population_count(mB))

        kv = tuple(wink_v[pl.ds(q * 16, 16)] for q in range(4))
        iv = tuple(wini_v[pl.ds(q * 16, 16)] for q in range(4))
        ok = (zeros16,) * 4
        oi = (zeros16,) * 4

        def ex(i, carry):
            kv, iv, ok, oi = carry
            mv = jnp.maximum(jnp.maximum(kv[0], kv[1]),
                             jnp.maximum(kv[2], kv[3]))
            mk = jnp.max(mv)
            mks = _bc(mk)
            big = _bc(jnp.int32(ROW_LEN))
            cands = [jnp.where(kv[q] == mks, iv[q], big) for q in range(4)]
            mi = jnp.min(jnp.minimum(jnp.minimum(cands[0], cands[1]),
                                     jnp.minimum(cands[2], cands[3])))
            mis = _bc(mi)
            i_s = _bc(i)
            slot = tuple((lane + 16 * q) == i_s for q in range(4))
            ok = tuple(jnp.where(slot[q], mks, ok[q]) for q in range(4))
            oi = tuple(jnp.where(slot[q], mis, oi[q]) for q in range(4))
            kv = tuple(
                jnp.where((kv[q] == mks) & (iv[q] == mis), _bc(KEY_MIN), kv[q])
                for q in range(4))
            return kv, iv, ok, oi

        _, _, ok, oi = lax.fori_loop(0, TOPK, ex, (kv, iv, ok, oi))

        for q in range(4):
            outv_v[pl.ds(q * 16, 16)] = _unkey(ok[q])
            outi_v[pl.ds(q * 16, 16)] = oi[q]
        pltpu.sync_copy(outv_v, vals_hbm.at[row])
        pltpu.sync_copy(outi_v, idx_hbm.at[row])

        return jnp.where(kstar < KEY_MIN + MARGIN_ROW,
                         jnp.int32(KEY_MIN), kstar - MARGIN_ROW)

    row_copy(row0).start()
    row_copy(row0).wait()

    @plsc.parallel_loop(0, NVREG, unroll=UA, carry=_bc(jnp.int32(KEY_MIN)))
    def mx(j, m):
        return jnp.maximum(m, _key(row_v[pl.ds(j * 16, 16)]))

    maxk = jnp.max(mx)
    t0 = jnp.where(maxk < KEY_MIN + MARGIN_BOOT,
                   jnp.int32(KEY_MIN), maxk - MARGIN_BOOT)
    row_copy(row0).start()

    lax.fori_loop(0, ROWS_PER_W, do_row, t0)
    row_copy(row0).wait()


@jax.jit
def _sc_topk(x_flat):
    f = pl.kernel(
        _sc_topk_body,
        out_type=[
            jax.ShapeDtypeStruct((ROWS, TOPK), jnp.float32),
            jax.ShapeDtypeStruct((ROWS, TOPK), jnp.int32),
        ],
        mesh=plsc.VectorSubcoreMesh(core_axis_name="c", subcore_axis_name="s",
                                    num_cores=2, num_subcores=16),
        scratch_types=[
            pltpu.VMEM((ROW_LEN,), jnp.float32),
            pltpu.VMEM((ROW_LEN + 16 * UB,), jnp.int32),
            pltpu.VMEM((ROW_LEN + 16 * UB,), jnp.int32),
            pltpu.VMEM((TOPK,), jnp.int32),
            pltpu.VMEM((TOPK,), jnp.int32),
            pltpu.VMEM((TOPK,), jnp.int32),
            pltpu.VMEM((TOPK,), jnp.float32),
            pltpu.SemaphoreType.DMA,
        ],
        compiler_params=pltpu.CompilerParams(needs_layout_passes=False),
    )
    return f(x_flat)


def kernel(pool_score):
    b0, b1, n = pool_score.shape
    x2 = pool_score.reshape(b0 * b1, n)
    vals, idx = _sc_topk(x2)
    return (vals.reshape(b0, b1, TOPK), idx.reshape(b0, b1, TOPK))


# speedup vs baseline: 2.6191x; 1.0317x over previous
import jax
import jax.numpy as jnp
import numpy as np
from jax import lax
from jax.experimental import pallas as pl
from jax.experimental.pallas import tpu as pltpu
from jax.experimental.pallas import tpu_sc as plsc

TOPK = 64
ROW_LEN = 32768
ROWS = 1024
NVREG = ROW_LEN // 16
ROWS_PER_W = ROWS // 32
MASK7F = np.int32(0x7FFFFFFF)
KEY_MIN = np.int32(-0x80000000)
MARGIN_ROW = np.int32(0x400000)
MARGIN_BOOT = np.int32(0x800000)
KEY_NEG_INF = np.int32(-2139095041)
UA = 16
UB = 4


def _key(x):
    b = plsc.bitcast(x, jnp.int32)
    return jnp.where(b < 0, b ^ MASK7F, b)


def _unkey(k):
    b = jnp.where(k < 0, k ^ MASK7F, k)
    return plsc.bitcast(b, jnp.float32)


def _bc(s, n=16):
    return lax.broadcast(s, (n,))


def _sc_topk_body(x_hbm, vals_hbm, idx_hbm, row_v, cand_v, candk_v,
                  wink_v, wini_v, outi_v, outv_v, dma_sem):
    wid = lax.axis_index("s") * 2 + lax.axis_index("c")
    lane = lax.iota(jnp.int32, 16)
    ones = jnp.ones((16,), jnp.int32)
    zeros16 = jnp.zeros((16,), jnp.int32)
    row0 = wid * ROWS_PER_W

    def row_copy(row):
        return pltpu.make_async_copy(x_hbm.at[row], row_v, dma_sem)

    def count_cmp(thresh, n_grp, strict):
        ts = _bc(thresh)

        @plsc.parallel_loop(0, n_grp * UB, unroll=UB, carry=zeros16)
        def cb(g, acc):
            k = candk_v[pl.ds(g * 16, 16)]
            m = (k > ts) if strict else (k >= ts)
            return acc + jnp.where(m, ones, zeros16)

        return jnp.sum(cb)

    def do_row(r, t_pred):
        row = row0 + r
        row_copy(row).wait()

        def cc(c):
            return c[0] < TOPK

        def cbody(c):
            _, t_try = c
            tf = _unkey(_bc(jnp.maximum(t_try, KEY_NEG_INF)))

            @plsc.parallel_loop(0, NVREG, unroll=UA, carry=zeros16)
            def pb(j, off_s):
                x = row_v[pl.ds(j * 16, 16)]
                m = x >= tf
                mi = jnp.where(m, ones, zeros16)
                pos = plsc.cumsum(mi) - 1 + off_s
                plsc.store_scatter(cand_v, [pos], j * 16 + lane, mask=m)
                return off_s + plsc.all_reduce_population_count(m)

            n = jnp.max(pb)
            return n, jnp.where(n < TOPK, jnp.int32(KEY_MIN), t_try)

        cand_n, _ = lax.while_loop(cc, cbody, (jnp.int32(0), t_pred))

        n_grp = (cand_n + (16 * UB - 1)) // (16 * UB)

        for u in range(UB):
            plsc.store_scatter(cand_v, [_bc(cand_n) + lane + 16 * u], zeros16)

        @plsc.parallel_loop(0, n_grp * UB, unroll=UB, carry=jnp.int32(0))
        def kf(v, acc):
            idx = cand_v[pl.ds(v * 16, 16)]
            b = plsc.bitcast(plsc.load_gather(row_v, [idx]), jnp.int32)
            candk_v[pl.ds(v * 16, 16)] = jnp.where(b < 0, b ^ MASK7F, b)
            return acc

        nxt = jnp.where(r < ROWS_PER_W - 1, row + 1, row0)
        row_copy(nxt).start()

        for u in range(UB):
            plsc.store_scatter(
                candk_v, [_bc(cand_n) + lane + 16 * u], _bc(KEY_MIN))

        def bsearch(i, t):
            bit = jnp.int32(1) << (jnp.int32(31) - i)
            tt = t + bit
            c = count_cmp(tt, n_grp, False)
            return jnp.where(c >= TOPK, tt, t)

        kstar = lax.fori_loop(0, 32, bsearch, jnp.int32(KEY_MIN))
        c_above = count_cmp(kstar, n_grp, True)
        k_eq = TOPK - c_above

        ks_s = _bc(kstar)
        cns = _bc(cand_n)
        nvr = (cand_n + 15) // 16
        k_eq_s = _bc(k_eq)

        @plsc.parallel_loop(0, nvr, carry=(zeros16, zeros16))
        def wc(v, carry):
            woff_s, eq_s = carry
            idxs = cand_v[pl.ds(v * 16, 16)]
            k = candk_v[pl.ds(v * 16, 16)]
            valid = (v * 16 + lane) < cns
            mA = (k > ks_s) & valid
            mB = (k == ks_s) & valid
            mBi = jnp.where(mB, ones, zeros16)
            csB = plsc.cumsum(mBi)
            mBsel = mB & ((csB + eq_s) <= k_eq_s)
            win = mA | mBsel
            wi = jnp.where(win, ones, zeros16)
            pos = plsc.cumsum(wi) - 1 + woff_s
            plsc.store_scatter(wini_v, [pos], idxs, mask=win)
            plsc.store_scatter(wink_v, [pos], k, mask=win)
            return (woff_s + plsc.all_reduce_population_count(win),
                    eq_s + plsc.all_reduce_population_count(mB))

        kv = tuple(wink_v[pl.ds(q * 16, 16)] for q in range(4))
        iv = tuple(wini_v[pl.ds(q * 16, 16)] for q in range(4))
        ok = (zeros16,) * 4
        oi = (zeros16,) * 4

        def ex(i, carry):
            kv, iv, ok, oi = carry
            mv = jnp.maximum(jnp.maximum(kv[0], kv[1]),
                             jnp.maximum(kv[2], kv[3]))
            mk = jnp.max(mv)
            mks = _bc(mk)
            big = _bc(jnp.int32(ROW_LEN))
            cands = [jnp.where(kv[q] == mks, iv[q], big) for q in range(4)]
            mi = jnp.min(jnp.minimum(jnp.minimum(cands[0], cands[1]),
                                     jnp.minimum(cands[2], cands[3])))
            mis = _bc(mi)
            i_s = _bc(i)
            slot = tuple((lane + 16 * q) == i_s for q in range(4))
            ok = tuple(jnp.where(slot[q], mks, ok[q]) for q in range(4))
            oi = tuple(jnp.where(slot[q], mis, oi[q]) for q in range(4))
            kv = tuple(
                jnp.where((kv[q] == mks) & (iv[q] == mis), _bc(KEY_MIN), kv[q])
                for q in range(4))
            return kv, iv, ok, oi

        _, _, ok, oi = lax.fori_loop(0, TOPK, ex, (kv, iv, ok, oi))

        for q in range(4):
            outv_v[pl.ds(q * 16, 16)] = _unkey(ok[q])
            outi_v[pl.ds(q * 16, 16)] = oi[q]
        pltpu.sync_copy(outv_v, vals_hbm.at[row])
        pltpu.sync_copy(outi_v, idx_hbm.at[row])

        return jnp.where(kstar < KEY_MIN + MARGIN_ROW,
                         jnp.int32(KEY_MIN), kstar - MARGIN_ROW)

    row_copy(row0).start()
    row_copy(row0).wait()

    @plsc.parallel_loop(0, NVREG, unroll=UA, carry=_bc(jnp.int32(KEY_MIN)))
    def mx(j, m):
        return jnp.maximum(m, _key(row_v[pl.ds(j * 16, 16)]))

    maxk = jnp.max(mx)
    t0 = jnp.where(maxk < KEY_MIN + MARGIN_BOOT,
                   jnp.int32(KEY_MIN), maxk - MARGIN_BOOT)
    row_copy(row0).start()

    lax.fori_loop(0, ROWS_PER_W, do_row, t0)
    row_copy(row0).wait()


@jax.jit
def _sc_topk(x_flat):
    f = pl.kernel(
        _sc_topk_body,
        out_type=[
            jax.ShapeDtypeStruct((ROWS, TOPK), jnp.float32),
            jax.ShapeDtypeStruct((ROWS, TOPK), jnp.int32),
        ],
        mesh=plsc.VectorSubcoreMesh(core_axis_name="c", subcore_axis_name="s",
                                    num_cores=2, num_subcores=16),
        scratch_types=[
            pltpu.VMEM((ROW_LEN,), jnp.float32),
            pltpu.VMEM((ROW_LEN + 16 * UB,), jnp.int32),
            pltpu.VMEM((ROW_LEN + 16 * UB,), jnp.int32),
            pltpu.VMEM((TOPK,), jnp.int32),
            pltpu.VMEM((TOPK,), jnp.int32),
            pltpu.VMEM((TOPK,), jnp.int32),
            pltpu.VMEM((TOPK,), jnp.float32),
            pltpu.SemaphoreType.DMA,
        ],
        compiler_params=pltpu.CompilerParams(needs_layout_passes=False),
    )
    return f(x_flat)


def kernel(pool_score):
    b0, b1, n = pool_score.shape
    x2 = pool_score.reshape(b0 * b1, n)
    vals, idx = _sc_topk(x2)
    return (vals.reshape(b0, b1, TOPK), idx.reshape(b0, b1, TOPK))
